# Initial kernel scaffold; baseline (speedup 1.0000x reference)
import jax
import jax.numpy as jnp
from jax.experimental import pallas as pl


def kernel(x, edge_index, edge_weight, batch, nrows, ncols, conv2d_w, bn2d_g, bn2d_b, gcn_w, gcn_b, bn_g, bn_b, Wq, bq, Wk, bk, Wv, bv, We, Wskip, bskip, lin_w, lin_b):
    raise NotImplementedError("write your pallas kernel here")


# speedup vs baseline: 1.4318x; 1.4318x over previous
import jax
import jax.numpy as jnp
from jax import lax
from jax.experimental import pallas as pl
from jax.experimental.pallas import tpu as pltpu

_N = 10000
_C = 128
_RS = 0.08838834764831845


def _elu(x):
    return jnp.where(x > 0, x, 0.1 * (jnp.exp(x) - 1.0))


def _bn_rows(h, g, b):
    mu = jnp.mean(h, axis=0, keepdims=True)
    var = jnp.mean((h - mu) ** 2, axis=0, keepdims=True)
    return (h - mu) / jnp.sqrt(var + 1e-5) * g[None] + b[None]


def _k1_body(xpad_ref, w_ref, g_ref, b_ref, gcnw_ref, deg_ref,
             hlin_ref, dinv_ref):
    acc = jnp.zeros((_N, _C), jnp.float32)
    for di in range(3):
        for dj in range(3):
            xs = xpad_ref[di:di + 100, dj:dj + 100, :].reshape(_N, _C)
            acc = acc + jnp.dot(xs, w_ref[di, dj],
                                preferred_element_type=jnp.float32)
    h = _bn_rows(acc, g_ref[...], b_ref[...])
    h = _elu(h)
    hlin_ref[...] = jnp.dot(h, gcnw_ref[...].T,
                            preferred_element_type=jnp.float32)
    dinv_ref[...] = lax.rsqrt(deg_ref[...] + 2.0)


def _k1(xpad, w, g, b, gcnw, deg_raw):
    return pl.pallas_call(
        _k1_body,
        out_shape=(jax.ShapeDtypeStruct((_N, _C), jnp.float32),
                   jax.ShapeDtypeStruct((_N,), jnp.float32)),
    )(xpad, w, g, b, gcnw, deg_raw)


def _k2_body(u_ref, hlin_ref, dinv_ref, gcnb_ref, g_ref, b_ref,
             wq_ref, bq_ref, wk_ref, bk_ref, wv_ref, bv_ref, wepad_ref,
             wskip_ref, bskip_ref,
             q_ref, k_ref, v_ref, qe_ref, hskip_ref):
    dinv = dinv_ref[...]
    h1 = (u_ref[...] + (2.0 * dinv * dinv)[:, None] * hlin_ref[...]
          + gcnb_ref[...][None])
    h1 = _elu(_bn_rows(h1, g_ref[...], b_ref[...]))
    q = jnp.dot(h1, wq_ref[...].T, preferred_element_type=jnp.float32) + bq_ref[...][None]
    k = jnp.dot(h1, wk_ref[...].T, preferred_element_type=jnp.float32) + bk_ref[...][None]
    v = jnp.dot(h1, wv_ref[...].T, preferred_element_type=jnp.float32) + bv_ref[...][None]
    q_ref[...] = q
    k_ref[...] = k
    v_ref[...] = v
    qe_ref[...] = jnp.dot(q, wepad_ref[...], preferred_element_type=jnp.float32)
    hskip_ref[...] = (jnp.dot(h1, wskip_ref[...].T,
                              preferred_element_type=jnp.float32)
                      + bskip_ref[...][None])


def _k2(u, hlin, dinv, gcnb, g, b, wq, bq, wk, bk, wv, bv, wepad, wskip, bskip):
    sh = jax.ShapeDtypeStruct((_N, _C), jnp.float32)
    return pl.pallas_call(
        _k2_body,
        out_shape=(sh, sh, sh, sh, sh),
    )(u, hlin, dinv, gcnb, g, b, wq, bq, wk, bk, wv, bv, wepad, wskip, bskip)


def _k3_body(ut_ref, aux_ref, wemat_ref, hskip_ref, g_ref, b_ref,
             linw_ref, linb_ref, out_ref):
    aux = aux_ref[...]
    ssum = aux[:, 0:1]
    edge_part = jnp.dot(aux, wemat_ref[...], preferred_element_type=jnp.float32)
    out_t = (ut_ref[...] + edge_part) / (ssum + 1e-16)
    h2 = out_t + hskip_ref[...]
    h2 = _bn_rows(h2, g_ref[...], b_ref[...])
    out_ref[...] = (jnp.dot(h2, linw_ref[...].T,
                            preferred_element_type=jnp.float32)
                    + linb_ref[...][None])


def _k3(ut, aux, wemat, hskip, g, b, linw, linb):
    return pl.pallas_call(
        _k3_body,
        out_shape=jax.ShapeDtypeStruct((_N, _C), jnp.float32),
    )(ut, aux, wemat, hskip, g, b, linw, linb)


def kernel(x, edge_index, edge_weight, batch, nrows, ncols, conv2d_w, bn2d_g,
           bn2d_b, gcn_w, gcn_b, bn_g, bn_b, Wq, bq, Wk, bk, Wv, bv, We,
           Wskip, bskip, lin_w, lin_b):
    src = edge_index[0]
    dst = edge_index[1]
    ew = edge_weight[:, 1]

    x2 = x + jnp.asarray(nrows * ncols - _N, jnp.float32)
    xpad = jnp.pad(x2.reshape(100, 100, _C), ((1, 1), (1, 1), (0, 0)))

    deg_raw = jax.ops.segment_sum(ew, dst, num_segments=_N)

    hlin, dinv = _k1(xpad, conv2d_w, bn2d_g, bn2d_b, gcn_w, deg_raw)

    norm = dinv[src] * ew * dinv[dst]
    u_gcn = jax.ops.segment_sum(norm[:, None] * hlin[src], dst,
                                num_segments=_N)

    wepad = jnp.zeros((_C, _C), jnp.float32).at[:, :2].set(We)
    q, k, v, qe, hskip = _k2(u_gcn, hlin, dinv, gcn_b, bn_g, bn_b,
                             Wq, bq, Wk, bk, Wv, bv, wepad, Wskip, bskip)

    alpha = ((q[dst] * k[src]).sum(-1)
             + (qe[:, :2][dst] * edge_weight).sum(-1)) * _RS
    a = jnp.exp(alpha)
    ssum = jax.ops.segment_sum(a, dst, num_segments=_N)
    ut = jax.ops.segment_sum(a[:, None] * v[src], dst, num_segments=_N)
    s2 = jax.ops.segment_sum(a[:, None] * edge_weight, dst, num_segments=_N)

    aux = jnp.zeros((_N, _C), jnp.float32)
    aux = aux.at[:, 0].set(ssum).at[:, 1:3].set(s2)
    wemat = jnp.zeros((_C, _C), jnp.float32).at[1, :].set(We[:, 0]).at[2, :].set(We[:, 1])

    return _k3(ut, aux, wemat, hskip, bn_g, bn_b, lin_w, lin_b)


# speedup vs baseline: 2.2579x; 1.5770x over previous
import jax
import jax.numpy as jnp
from jax import lax
from jax.experimental import pallas as pl
from jax.experimental.pallas import tpu as pltpu
from jax.experimental.pallas import tpu_sc as plsc

_N = 10000
_C = 128
_RS = 0.08838834764831845

_NC, _NS, _L = 2, 16, 16
_NW = _NC * _NS
_NPAD = 10240
_E = 320000
_EW = 64
_ERW = 5120
_EPAD = _ERW * _EW
_ERT = _ERW // _NW
_ZS = _NPAD // _NS

_sc_mesh = plsc.VectorSubcoreMesh(core_axis_name="c", subcore_axis_name="s",
                                  num_cores=_NC, num_subcores=_NS)
_sc_params = pltpu.CompilerParams(needs_layout_passes=False)


def _elu(x):
    return jnp.where(x > 0, x, 0.1 * (jnp.exp(x) - 1.0))


def _bn_rows(h, g, b):
    mu = jnp.mean(h, axis=0, keepdims=True)
    var = jnp.mean((h - mu) ** 2, axis=0, keepdims=True)
    return (h - mu) / jnp.sqrt(var + 1e-5) * g[None] + b[None]


def _sca_body(dstp, ewp, zerov, out, idx_v, val_v, deg_sh):
    c = lax.axis_index("c")
    s = lax.axis_index("s")
    wid = s * _NC + c
    pltpu.sync_copy(zerov.at[pl.ds(s * _ZS, _ZS)],
                    deg_sh.at[pl.ds(s * _ZS, _ZS)])
    plsc.subcore_barrier()
    base = wid * _ERT

    def chunk(t, carry):
        pltpu.sync_copy(dstp.at[pl.ds(base + t * 32, 32)], idx_v)
        pltpu.sync_copy(ewp.at[pl.ds(base + t * 32, 32)], val_v)

        def row(j, cc):
            pltpu.sync_copy(val_v.at[j], deg_sh.at[idx_v.at[j]], add=True)
            return cc

        return lax.fori_loop(0, 32, row, carry)

    lax.fori_loop(0, _ERT // 32, chunk, 0)
    plsc.subcore_barrier()
    pltpu.sync_copy(deg_sh.at[pl.ds(s * _ZS, _ZS)],
                    out.at[c, pl.ds(s * _ZS, _ZS)])


def _sca(dstp, ewp, zerov):
    return pl.kernel(
        _sca_body,
        out_type=jax.ShapeDtypeStruct((_NC, _NPAD), jnp.float32),
        mesh=_sc_mesh,
        scratch_types=[pltpu.VMEM((32, _EW), jnp.int32),
                       pltpu.VMEM((32, _EW), jnp.float32),
                       pltpu.VMEM_SHARED((_NPAD,), jnp.float32)],
        compiler_params=_sc_params,
    )(dstp, ewp, zerov)


def _scb_body(srcp, dstp, ewp, hlin2p, zerot, ub,
              src_b, dst_b, ew_b, rows_v, acc_sh):
    c = lax.axis_index("c")
    s = lax.axis_index("s")
    wid = s * _NC + c
    pltpu.sync_copy(zerot.at[pl.ds(s * _ZS, _ZS)],
                    acc_sh.at[pl.ds(s * _ZS, _ZS)])
    plsc.subcore_barrier()
    base = wid * _ERT
    iota = lax.iota(jnp.int32, 16)

    def block(bi, carry):
        r0 = base + bi * 8
        pltpu.sync_copy(srcp.at[pl.ds(r0, 8)], src_b)
        pltpu.sync_copy(dstp.at[pl.ds(r0, 8)], dst_b)
        pltpu.sync_copy(ewp.at[pl.ds(r0, 8)], ew_b)

        def chunk(rr, cc):
            pltpu.sync_copy(hlin2p.at[src_b.at[rr]], rows_v)
            ew = [ew_b[rr, pl.ds(g * 16, 16)] for g in range(4)]

            def feat(f, c2):
                fv = jnp.full((16,), f, jnp.int32)
                for g in range(4):
                    row16 = g * 16 + iota
                    hv = plsc.load_gather(rows_v, [row16, fv])
                    plsc.store_scatter(rows_v, [row16, fv], hv * ew[g])
                return c2

            lax.fori_loop(0, 128, feat, 0)
            pltpu.sync_copy(rows_v, acc_sh.at[dst_b.at[rr]], add=True)
            return cc

        return lax.fori_loop(0, 8, chunk, carry)

    lax.fori_loop(0, _ERT // 8, block, 0)
    plsc.subcore_barrier()
    pltpu.sync_copy(acc_sh.at[pl.ds(s * _ZS, _ZS)],
                    ub.at[c, pl.ds(s * _ZS, _ZS)])


def _scb(srcp, dstp, ewp, hlin2p, zerot):
    return pl.kernel(
        _scb_body,
        out_type=jax.ShapeDtypeStruct((_NC, _NPAD, _C), jnp.float32),
        mesh=_sc_mesh,
        scratch_types=[pltpu.VMEM((8, _EW), jnp.int32),
                       pltpu.VMEM((8, _EW), jnp.int32),
                       pltpu.VMEM((8, _EW), jnp.float32),
                       pltpu.VMEM((_EW, _C), jnp.float32),
                       pltpu.VMEM_SHARED((_NPAD, _C), jnp.float32)],
        compiler_params=_sc_params,
    )(srcp, dstp, ewp, hlin2p, zerot)


def _scc_body(srcp, dstp, e0p, e1p, qp, kvp, we0in, we1in, zerot, zerov,
              utb, ssb,
              src_b, dst_b, e0_b, e1_b, q_rows, kv_rows, vs_buf, a_buf,
              we0_v, we1_v, acc_sh, ss_sh):
    c = lax.axis_index("c")
    s = lax.axis_index("s")
    wid = s * _NC + c
    pltpu.sync_copy(we0in, we0_v)
    pltpu.sync_copy(we1in, we1_v)
    pltpu.sync_copy(zerot.at[pl.ds(s * _ZS, _ZS)],
                    acc_sh.at[pl.ds(s * _ZS, _ZS)])
    pltpu.sync_copy(zerov.at[pl.ds(s * _ZS, _ZS)],
                    ss_sh.at[pl.ds(s * _ZS, _ZS)])
    plsc.subcore_barrier()
    base = wid * _ERT
    iota = lax.iota(jnp.int32, 16)
    z16 = jnp.zeros((16,), jnp.float32)

    def block(bi, carry):
        r0 = base + bi * 8
        pltpu.sync_copy(srcp.at[pl.ds(r0, 8)], src_b)
        pltpu.sync_copy(dstp.at[pl.ds(r0, 8)], dst_b)
        pltpu.sync_copy(e0p.at[pl.ds(r0, 8)], e0_b)
        pltpu.sync_copy(e1p.at[pl.ds(r0, 8)], e1_b)

        def chunk(rr, cc):
            pltpu.sync_copy(qp.at[dst_b.at[rr]], q_rows)
            pltpu.sync_copy(kvp.at[src_b.at[rr]], kv_rows)
            ew0 = [e0_b[rr, pl.ds(g * 16, 16)] for g in range(4)]
            ew1 = [e1_b[rr, pl.ds(g * 16, 16)] for g in range(4)]

            def feat(f, al):
                fv = jnp.full((16,), f, jnp.int32)
                w0 = plsc.load_gather(we0_v, [fv])
                w1 = plsc.load_gather(we1_v, [fv])
                out = []
                for g in range(4):
                    row16 = g * 16 + iota
                    qa = plsc.load_gather(q_rows, [row16, fv])
                    ka = plsc.load_gather(kv_rows, [row16, fv])
                    out.append(al[g] + qa * (ka + w0 * ew0[g] + w1 * ew1[g]))
                return tuple(out)

            al = lax.fori_loop(0, 128, feat, (z16, z16, z16, z16))
            a16 = [jnp.exp(al[g] * _RS) for g in range(4)]
            for g in range(4):
                a_buf[0, pl.ds(g * 16, 16)] = a16[g]

            def feat2(f, c2):
                fv = jnp.full((16,), f, jnp.int32)
                w0 = plsc.load_gather(we0_v, [fv])
                w1 = plsc.load_gather(we1_v, [fv])
                for g in range(4):
                    row16 = g * 16 + iota
                    va = plsc.load_gather(kv_rows, [row16, fv + 128])
                    plsc.store_scatter(
                        vs_buf, [row16, fv],
                        (va + w0 * ew0[g] + w1 * ew1[g]) * a16[g])
                return c2

            lax.fori_loop(0, 128, feat2, 0)
            pltpu.sync_copy(vs_buf, acc_sh.at[dst_b.at[rr]], add=True)
            pltpu.sync_copy(a_buf.at[0], ss_sh.at[dst_b.at[rr]], add=True)
            return cc

        return lax.fori_loop(0, 8, chunk, carry)

    lax.fori_loop(0, _ERT // 8, block, 0)
    plsc.subcore_barrier()
    pltpu.sync_copy(acc_sh.at[pl.ds(s * _ZS, _ZS)],
                    utb.at[c, pl.ds(s * _ZS, _ZS)])
    pltpu.sync_copy(ss_sh.at[pl.ds(s * _ZS, _ZS)],
                    ssb.at[c, pl.ds(s * _ZS, _ZS)])


def _scc(srcp, dstp, e0p, e1p, qp, kvp, we0in, we1in, zerot, zerov):
    return pl.kernel(
        _scc_body,
        out_type=(jax.ShapeDtypeStruct((_NC, _NPAD, _C), jnp.float32),
                  jax.ShapeDtypeStruct((_NC, _NPAD), jnp.float32)),
        mesh=_sc_mesh,
        scratch_types=[pltpu.VMEM((8, _EW), jnp.int32),
                       pltpu.VMEM((8, _EW), jnp.int32),
                       pltpu.VMEM((8, _EW), jnp.float32),
                       pltpu.VMEM((8, _EW), jnp.float32),
                       pltpu.VMEM((_EW, _C), jnp.float32),
                       pltpu.VMEM((_EW, 2 * _C), jnp.float32),
                       pltpu.VMEM((_EW, _C), jnp.float32),
                       pltpu.VMEM((1, _EW), jnp.float32),
                       pltpu.VMEM((_C,), jnp.float32),
                       pltpu.VMEM((_C,), jnp.float32),
                       pltpu.VMEM_SHARED((_NPAD, _C), jnp.float32),
                       pltpu.VMEM_SHARED((_NPAD,), jnp.float32)],
        compiler_params=_sc_params,
    )(srcp, dstp, e0p, e1p, qp, kvp, we0in, we1in, zerot, zerov)


def _k1_body(xpad_ref, w_ref, g_ref, b_ref, gcnw_ref, deg0_ref, deg1_ref,
             hlin_ref, hlin2_ref, dinv_ref):
    acc = jnp.zeros((_N, _C), jnp.float32)
    for di in range(3):
        for dj in range(3):
            xs = xpad_ref[di:di + 100, dj:dj + 100, :].reshape(_N, _C)
            acc = acc + jnp.dot(xs, w_ref[di, dj],
                                preferred_element_type=jnp.float32)
    h = _bn_rows(acc, g_ref[...], b_ref[...])
    h = _elu(h)
    hlin = jnp.dot(h, gcnw_ref[...].T, preferred_element_type=jnp.float32)
    dinv = lax.rsqrt(deg0_ref[...] + deg1_ref[...] + 2.0)
    hlin_ref[...] = hlin
    hlin2_ref[...] = dinv[:, None] * hlin
    dinv_ref[...] = dinv


def _k1(xpad, w, g, b, gcnw, deg0, deg1):
    sh = jax.ShapeDtypeStruct((_N, _C), jnp.float32)
    return pl.pallas_call(
        _k1_body,
        out_shape=(sh, sh, jax.ShapeDtypeStruct((_N,), jnp.float32)),
    )(xpad, w, g, b, gcnw, deg0, deg1)


def _k2_body(u0_ref, u1_ref, hlin_ref, dinv_ref, gcnb_ref, g_ref, b_ref,
             wq_ref, bq_ref, wk_ref, bk_ref, wv_ref, bv_ref,
             wskip_ref, bskip_ref,
             q_ref, k_ref, v_ref, hskip_ref):
    dinv = dinv_ref[...]
    h1 = (dinv[:, None] * (u0_ref[...] + u1_ref[...])
          + (2.0 * dinv * dinv)[:, None] * hlin_ref[...]
          + gcnb_ref[...][None])
    h1 = _elu(_bn_rows(h1, g_ref[...], b_ref[...]))
    q_ref[...] = jnp.dot(h1, wq_ref[...].T,
                         preferred_element_type=jnp.float32) + bq_ref[...][None]
    k_ref[...] = jnp.dot(h1, wk_ref[...].T,
                         preferred_element_type=jnp.float32) + bk_ref[...][None]
    v_ref[...] = jnp.dot(h1, wv_ref[...].T,
                         preferred_element_type=jnp.float32) + bv_ref[...][None]
    hskip_ref[...] = (jnp.dot(h1, wskip_ref[...].T,
                              preferred_element_type=jnp.float32)
                      + bskip_ref[...][None])


def _k2(u0, u1, hlin, dinv, gcnb, g, b, wq, bq, wk, bk, wv, bv, wskip, bskip):
    sh = jax.ShapeDtypeStruct((_N, _C), jnp.float32)
    return pl.pallas_call(
        _k2_body,
        out_shape=(sh, sh, sh, sh),
    )(u0, u1, hlin, dinv, gcnb, g, b, wq, bq, wk, bk, wv, bv, wskip, bskip)


def _k3_body(ut0_ref, ut1_ref, ss0_ref, ss1_ref, hskip_ref,
             g_ref, b_ref, linw_ref, linb_ref, out_ref):
    ssum = ss0_ref[...] + ss1_ref[...]
    out_t = (ut0_ref[...] + ut1_ref[...]) / (ssum + 1e-16)[:, None]
    h2 = out_t + hskip_ref[...]
    h2 = _bn_rows(h2, g_ref[...], b_ref[...])
    out_ref[...] = (jnp.dot(h2, linw_ref[...].T,
                            preferred_element_type=jnp.float32)
                    + linb_ref[...][None])


def _k3(ut0, ut1, ss0, ss1, hskip, g, b, linw, linb):
    return pl.pallas_call(
        _k3_body,
        out_shape=jax.ShapeDtypeStruct((_N, _C), jnp.float32),
    )(ut0, ut1, ss0, ss1, hskip, g, b, linw, linb)


def kernel(x, edge_index, edge_weight, batch, nrows, ncols, conv2d_w, bn2d_g,
           bn2d_b, gcn_w, gcn_b, bn_g, bn_b, Wq, bq, Wk, bk, Wv, bv, We,
           Wskip, bskip, lin_w, lin_b):
    src = edge_index[0]
    dst = edge_index[1]

    npad_e = _EPAD - _E
    srcp = jnp.pad(src, (0, npad_e)).reshape(_ERW, _EW)
    dstp = jnp.pad(dst, (0, npad_e), constant_values=_N).reshape(_ERW, _EW)
    ewpad = jnp.pad(edge_weight, ((0, npad_e), (0, 0)))
    ew0p = ewpad[:, 0].reshape(_ERW, _EW)
    ew1p = ewpad[:, 1].reshape(_ERW, _EW)

    zerot = jnp.zeros((_NPAD, _C), jnp.float32)
    zerov = jnp.zeros((_NPAD,), jnp.float32)

    x2 = x + jnp.asarray(nrows * ncols - _N, jnp.float32)
    xpad = jnp.pad(x2.reshape(100, 100, _C), ((1, 1), (1, 1), (0, 0)))

    degb = _sca(dstp, ew1p, zerov)
    hlin, hlin2, dinv = _k1(xpad, conv2d_w, bn2d_g, bn2d_b, gcn_w,
                            degb[0, :_N], degb[1, :_N])

    hlin2p = jnp.pad(hlin2, ((0, _NPAD - _N), (0, 0)))
    ub = _scb(srcp, dstp, ew1p, hlin2p, zerot)

    q, k, v, hskip = _k2(ub[0, :_N], ub[1, :_N], hlin, dinv, gcn_b,
                         bn_g, bn_b, Wq, bq, Wk, bk, Wv, bv, Wskip, bskip)

    qp = jnp.pad(q, ((0, _NPAD - _N), (0, 0)))
    kvp = jnp.pad(jnp.concatenate([k, v], axis=1), ((0, _NPAD - _N), (0, 0)))

    utb, ssb = _scc(srcp, dstp, ew0p, ew1p, qp, kvp, We[:, 0], We[:, 1],
                    zerot, zerov)

    return _k3(utb[0, :_N], utb[1, :_N], ssb[0, :_N], ssb[1, :_N],
               hskip, bn_g, bn_b, lin_w, lin_b)


# speedup vs baseline: 3.4382x; 1.5228x over previous
import jax
import jax.numpy as jnp
from jax import lax
from jax.experimental import pallas as pl
from jax.experimental.pallas import tpu as pltpu
from jax.experimental.pallas import tpu_sc as plsc

_N = 10000
_C = 128
_RS = 0.08838834764831845

_NC, _NS, _L = 2, 16, 16
_NW = _NC * _NS
_NPAD = 10240
_E = 320000
_EW = 32
_ERW = 10240
_EPAD = _ERW * _EW
_ERT = _ERW // _NW
_NCH = 16
_NBLK = _ERT // _NCH
_ZS = _NPAD // _NS

_sc_mesh = plsc.VectorSubcoreMesh(core_axis_name="c", subcore_axis_name="s",
                                  num_cores=_NC, num_subcores=_NS)
_sc_params = pltpu.CompilerParams(needs_layout_passes=False)


def _elu(x):
    return jnp.where(x > 0, x, 0.1 * (jnp.exp(x) - 1.0))


def _bn_rows(h, g, b):
    mu = jnp.mean(h, axis=0, keepdims=True)
    var = jnp.mean((h - mu) ** 2, axis=0, keepdims=True)
    return (h - mu) / jnp.sqrt(var + 1e-5) * g[None] + b[None]


def _sca_body(dstp, ewp, zerov, out, idx_v, val_v, deg_sh):
    c = lax.axis_index("c")
    s = lax.axis_index("s")
    wid = s * _NC + c
    pltpu.sync_copy(zerov.at[pl.ds(s * _ZS, _ZS)],
                    deg_sh.at[pl.ds(s * _ZS, _ZS)])
    plsc.subcore_barrier()
    base = wid * _ERT

    def chunk(t, carry):
        pltpu.sync_copy(dstp.at[pl.ds(base + t * 32, 32)], idx_v)
        pltpu.sync_copy(ewp.at[pl.ds(base + t * 32, 32)], val_v)

        def row(j, cc):
            pltpu.sync_copy(val_v.at[j], deg_sh.at[idx_v.at[j]], add=True)
            return cc

        return lax.fori_loop(0, 32, row, carry)

    lax.fori_loop(0, _ERT // 32, chunk, 0)
    plsc.subcore_barrier()
    pltpu.sync_copy(deg_sh.at[pl.ds(s * _ZS, _ZS)],
                    out.at[pl.ds(c * _NPAD + s * _ZS, _ZS)])


def _sca(dstp, ewp, zerov):
    return pl.kernel(
        _sca_body,
        out_type=jax.ShapeDtypeStruct((_NC * _NPAD,), jnp.float32),
        mesh=_sc_mesh,
        scratch_types=[pltpu.VMEM((32, _EW), jnp.int32),
                       pltpu.VMEM((32, _EW), jnp.float32),
                       pltpu.VMEM_SHARED((_NPAD,), jnp.float32)],
        compiler_params=_sc_params,
    )(dstp, ewp, zerov)


def _scb_body(srcp, dstp, ewp, hlin2p, zerot, ub,
              src_b, dst_b, ew_b, rows0, rows1, vsb0, vsb1,
              gs0, gs1, ss0, ss1, acc_sh):
    c = lax.axis_index("c")
    s = lax.axis_index("s")
    wid = s * _NC + c
    rows = [rows0, rows1]
    vsb = [vsb0, vsb1]
    gsem = [gs0, gs1]
    ssem = [ss0, ss1]
    pltpu.sync_copy(zerot.at[pl.ds(s * _ZS, _ZS)],
                    acc_sh.at[pl.ds(s * _ZS, _ZS)])
    plsc.subcore_barrier()
    base = wid * _ERT
    iota = lax.iota(jnp.int32, 16)

    def block(lb, carry):
        r0 = base + lb * _NCH
        pltpu.sync_copy(srcp.at[pl.ds(r0, _NCH)], src_b)
        pltpu.sync_copy(dstp.at[pl.ds(r0, _NCH)], dst_b)
        pltpu.sync_copy(ewp.at[pl.ds(r0, _NCH)], ew_b)
        for b in range(2):
            pltpu.async_copy(hlin2p.at[src_b.at[b]], rows[b], gsem[b])

        def pair(cp, cc):
            for b in range(2):
                ch = cp * 2 + b
                pltpu.make_async_copy(hlin2p.at[src_b.at[ch]],
                                      rows[b], gsem[b]).wait()

                @pl.when(ch >= 2)
                def _():
                    pltpu.make_async_copy(
                        vsb[b], acc_sh.at[dst_b.at[ch]], ssem[b]).wait()

                ew = [ew_b[ch, pl.ds(g * 16, 16)] for g in range(2)]

                def feat(f, c2):
                    fv = jnp.full((16,), f, jnp.int32)
                    for g in range(2):
                        row16 = g * 16 + iota
                        hv = plsc.load_gather(rows[b], [row16, fv])
                        plsc.store_scatter(vsb[b], [row16, fv], hv * ew[g])
                    return c2

                lax.fori_loop(0, 128, feat, 0)
                pltpu.async_copy(vsb[b], acc_sh.at[dst_b.at[ch]],
                                 ssem[b], add=True)

                @pl.when(ch + 2 < _NCH)
                def _():
                    pltpu.async_copy(hlin2p.at[src_b.at[ch + 2]],
                                     rows[b], gsem[b])

            return cc

        lax.fori_loop(0, _NCH // 2, pair, 0)
        for b in range(2):
            pltpu.make_async_copy(vsb[b], acc_sh.at[dst_b.at[b]],
                                  ssem[b]).wait()
        return carry

    lax.fori_loop(0, _NBLK, block, 0)
    plsc.subcore_barrier()
    pltpu.sync_copy(acc_sh.at[pl.ds(s * _ZS, _ZS)],
                    ub.at[c, pl.ds(s * _ZS, _ZS)])


def _scb(srcp, dstp, ewp, hlin2p, zerot):
    return pl.kernel(
        _scb_body,
        out_type=jax.ShapeDtypeStruct((_NC, _NPAD, _C), jnp.float32),
        mesh=_sc_mesh,
        scratch_types=[pltpu.VMEM((_NCH, _EW), jnp.int32),
                       pltpu.VMEM((_NCH, _EW), jnp.int32),
                       pltpu.VMEM((_NCH, _EW), jnp.float32),
                       pltpu.VMEM((_EW, _C), jnp.float32),
                       pltpu.VMEM((_EW, _C), jnp.float32),
                       pltpu.VMEM((_EW, _C), jnp.float32),
                       pltpu.VMEM((_EW, _C), jnp.float32),
                       pltpu.SemaphoreType.DMA,
                       pltpu.SemaphoreType.DMA,
                       pltpu.SemaphoreType.DMA,
                       pltpu.SemaphoreType.DMA,
                       pltpu.VMEM_SHARED((_NPAD, _C), jnp.float32)],
        compiler_params=_sc_params,
    )(srcp, dstp, ewp, hlin2p, zerot)


def _scc_body(srcp, dstp, e0p, e1p, qp, kvp, we0in, we1in, zerot, zerov,
              utb, ssb,
              src_b, dst_b, e0_b, e1_b, qr0, qr1, kvr0, kvr1, vsb0, vsb1,
              ab0, ab1, we0_v, we1_v,
              gs0, gs1, ss0, ss1, acc_sh, ss_sh):
    c = lax.axis_index("c")
    s = lax.axis_index("s")
    wid = s * _NC + c
    qr = [qr0, qr1]
    kvr = [kvr0, kvr1]
    vsb = [vsb0, vsb1]
    ab = [ab0, ab1]
    gsem = [gs0, gs1]
    ssem = [ss0, ss1]
    pltpu.sync_copy(we0in, we0_v)
    pltpu.sync_copy(we1in, we1_v)
    pltpu.sync_copy(zerot.at[pl.ds(s * _ZS, _ZS)],
                    acc_sh.at[pl.ds(s * _ZS, _ZS)])
    pltpu.sync_copy(zerov.at[pl.ds(s * _ZS, _ZS)],
                    ss_sh.at[pl.ds(s * _ZS, _ZS)])
    plsc.subcore_barrier()
    base = wid * _ERT
    iota = lax.iota(jnp.int32, 16)
    z16 = jnp.zeros((16,), jnp.float32)

    def block(lb, carry):
        r0 = base + lb * _NCH
        pltpu.sync_copy(srcp.at[pl.ds(r0, _NCH)], src_b)
        pltpu.sync_copy(dstp.at[pl.ds(r0, _NCH)], dst_b)
        pltpu.sync_copy(e0p.at[pl.ds(r0, _NCH)], e0_b)
        pltpu.sync_copy(e1p.at[pl.ds(r0, _NCH)], e1_b)
        for b in range(2):
            pltpu.async_copy(qp.at[dst_b.at[b]], qr[b], gsem[b])
            pltpu.async_copy(kvp.at[src_b.at[b]], kvr[b], gsem[b])

        def pair(cp, cc):
            for b in range(2):
                ch = cp * 2 + b
                pltpu.make_async_copy(qp.at[dst_b.at[ch]],
                                      qr[b], gsem[b]).wait()
                pltpu.make_async_copy(kvp.at[src_b.at[ch]],
                                      kvr[b], gsem[b]).wait()
                ew0 = [e0_b[ch, pl.ds(g * 16, 16)] for g in range(2)]
                ew1 = [e1_b[ch, pl.ds(g * 16, 16)] for g in range(2)]

                def feat(f, al):
                    fv = jnp.full((16,), f, jnp.int32)
                    w0g = plsc.load_gather(we0_v, [fv])
                    w0l = plsc.bitcast(lax.shift_left(w0g, 16), jnp.float32)
                    w0h = plsc.bitcast(w0g & jnp.int32(-65536), jnp.float32)
                    w1g = plsc.load_gather(we1_v, [fv])
                    w1l = plsc.bitcast(lax.shift_left(w1g, 16), jnp.float32)
                    w1h = plsc.bitcast(w1g & jnp.int32(-65536), jnp.float32)
                    out = []
                    for g in range(2):
                        row16 = g * 16 + iota
                        ql = plsc.load_gather(qr[b], [row16, fv * 2])
                        qh = plsc.load_gather(qr[b], [row16, fv * 2 + 1])
                        kg = plsc.load_gather(kvr[b], [row16, fv])
                        kl = plsc.bitcast(lax.shift_left(kg, 16),
                                          jnp.float32)
                        kh = plsc.bitcast(kg & jnp.int32(-65536),
                                          jnp.float32)
                        out.append(al[g]
                                   + ql * (kl + w0l * ew0[g] + w1l * ew1[g])
                                   + qh * (kh + w0h * ew0[g] + w1h * ew1[g]))
                    return tuple(out)

                al = lax.fori_loop(0, 64, feat, (z16, z16))
                a16 = [jnp.exp(al[g] * _RS) for g in range(2)]

                @pl.when(ch >= 2)
                def _():
                    pltpu.make_async_copy(
                        vsb[b], acc_sh.at[dst_b.at[ch]], ssem[b]).wait()
                    pltpu.make_async_copy(
                        ab[b].at[0], ss_sh.at[dst_b.at[ch]], ssem[b]).wait()

                for g in range(2):
                    ab[b][0, pl.ds(g * 16, 16)] = a16[g]

                def feat2(f, c2):
                    fv = jnp.full((16,), f, jnp.int32)
                    w0g = plsc.load_gather(we0_v, [fv])
                    w0l = plsc.bitcast(lax.shift_left(w0g, 16), jnp.float32)
                    w0h = plsc.bitcast(w0g & jnp.int32(-65536), jnp.float32)
                    w1g = plsc.load_gather(we1_v, [fv])
                    w1l = plsc.bitcast(lax.shift_left(w1g, 16), jnp.float32)
                    w1h = plsc.bitcast(w1g & jnp.int32(-65536), jnp.float32)
                    for g in range(2):
                        row16 = g * 16 + iota
                        vg = plsc.load_gather(kvr[b], [row16, fv + 64])
                        vl = plsc.bitcast(lax.shift_left(vg, 16),
                                          jnp.float32)
                        vh = plsc.bitcast(vg & jnp.int32(-65536),
                                          jnp.float32)
                        plsc.store_scatter(
                            vsb[b], [row16, fv * 2],
                            (vl + w0l * ew0[g] + w1l * ew1[g]) * a16[g])
                        plsc.store_scatter(
                            vsb[b], [row16, fv * 2 + 1],
                            (vh + w0h * ew0[g] + w1h * ew1[g]) * a16[g])
                    return c2

                lax.fori_loop(0, 64, feat2, 0)
                pltpu.async_copy(vsb[b], acc_sh.at[dst_b.at[ch]],
                                 ssem[b], add=True)
                pltpu.async_copy(ab[b].at[0], ss_sh.at[dst_b.at[ch]],
                                 ssem[b], add=True)

                @pl.when(ch + 2 < _NCH)
                def _():
                    pltpu.async_copy(qp.at[dst_b.at[ch + 2]], qr[b], gsem[b])
                    pltpu.async_copy(kvp.at[src_b.at[ch + 2]], kvr[b],
                                     gsem[b])

            return cc

        lax.fori_loop(0, _NCH // 2, pair, 0)
        for b in range(2):
            pltpu.make_async_copy(vsb[b], acc_sh.at[dst_b.at[b]],
                                  ssem[b]).wait()
            pltpu.make_async_copy(ab[b].at[0], ss_sh.at[dst_b.at[b]],
                                  ssem[b]).wait()
        return carry

    lax.fori_loop(0, _NBLK, block, 0)
    plsc.subcore_barrier()
    pltpu.sync_copy(acc_sh.at[pl.ds(s * _ZS, _ZS)],
                    utb.at[c, pl.ds(s * _ZS, _ZS)])
    pltpu.sync_copy(ss_sh.at[pl.ds(s * _ZS, _ZS)],
                    ssb.at[pl.ds(c * _NPAD + s * _ZS, _ZS)])


def _scc(srcp, dstp, e0p, e1p, qp, kvp, we0in, we1in, zerot, zerov):
    return pl.kernel(
        _scc_body,
        out_type=(jax.ShapeDtypeStruct((_NC, _NPAD, _C), jnp.float32),
                  jax.ShapeDtypeStruct((_NC * _NPAD,), jnp.float32)),
        mesh=_sc_mesh,
        scratch_types=[pltpu.VMEM((_NCH, _EW), jnp.int32),
                       pltpu.VMEM((_NCH, _EW), jnp.int32),
                       pltpu.VMEM((_NCH, _EW), jnp.float32),
                       pltpu.VMEM((_NCH, _EW), jnp.float32),
                       pltpu.VMEM((_EW, _C), jnp.float32),
                       pltpu.VMEM((_EW, _C), jnp.float32),
                       pltpu.VMEM((_EW, _C), jnp.int32),
                       pltpu.VMEM((_EW, _C), jnp.int32),
                       pltpu.VMEM((_EW, _C), jnp.float32),
                       pltpu.VMEM((_EW, _C), jnp.float32),
                       pltpu.VMEM((1, _EW), jnp.float32),
                       pltpu.VMEM((1, _EW), jnp.float32),
                       pltpu.VMEM((_C // 2,), jnp.int32),
                       pltpu.VMEM((_C // 2,), jnp.int32),
                       pltpu.SemaphoreType.DMA,
                       pltpu.SemaphoreType.DMA,
                       pltpu.SemaphoreType.DMA,
                       pltpu.SemaphoreType.DMA,
                       pltpu.VMEM_SHARED((_NPAD, _C), jnp.float32),
                       pltpu.VMEM_SHARED((_NPAD,), jnp.float32)],
        compiler_params=_sc_params,
    )(srcp, dstp, e0p, e1p, qp, kvp, we0in, we1in, zerot, zerov)


def _k1_body(xpad_ref, w_ref, g_ref, b_ref, gcnw_ref, deg0_ref, deg1_ref,
             hlin_ref, hlin2_ref, dinv_ref):
    acc = jnp.zeros((_N, _C), jnp.float32)
    for di in range(3):
        for dj in range(3):
            xs = xpad_ref[di:di + 100, dj:dj + 100, :].reshape(_N, _C)
            acc = acc + jnp.dot(xs, w_ref[di, dj],
                                preferred_element_type=jnp.float32)
    h = _bn_rows(acc, g_ref[...], b_ref[...])
    h = _elu(h)
    hlin = jnp.dot(h, gcnw_ref[...].T, preferred_element_type=jnp.float32)
    dinv = lax.rsqrt(deg0_ref[...] + deg1_ref[...] + 2.0)
    hlin_ref[...] = hlin
    hlin2_ref[...] = dinv[:, None] * hlin
    dinv_ref[...] = dinv


def _k1(xpad, w, g, b, gcnw, deg0, deg1):
    sh = jax.ShapeDtypeStruct((_N, _C), jnp.float32)
    return pl.pallas_call(
        _k1_body,
        out_shape=(sh, sh, jax.ShapeDtypeStruct((_N,), jnp.float32)),
    )(xpad, w, g, b, gcnw, deg0, deg1)


def _k2_body(u0_ref, u1_ref, hlin_ref, dinv_ref, gcnb_ref, g_ref, b_ref,
             wq_ref, bq_ref, wk_ref, bk_ref, wv_ref, bv_ref,
             wskip_ref, bskip_ref,
             q_ref, k_ref, v_ref, hskip_ref):
    dinv = dinv_ref[...]
    h1 = (dinv[:, None] * (u0_ref[...] + u1_ref[...])
          + (2.0 * dinv * dinv)[:, None] * hlin_ref[...]
          + gcnb_ref[...][None])
    h1 = _elu(_bn_rows(h1, g_ref[...], b_ref[...]))
    q_ref[...] = jnp.dot(h1, wq_ref[...].T,
                         preferred_element_type=jnp.float32) + bq_ref[...][None]
    k_ref[...] = (jnp.dot(h1, wk_ref[...].T,
                          preferred_element_type=jnp.float32)
                  + bk_ref[...][None]).astype(jnp.bfloat16)
    v_ref[...] = (jnp.dot(h1, wv_ref[...].T,
                          preferred_element_type=jnp.float32)
                  + bv_ref[...][None]).astype(jnp.bfloat16)
    hskip_ref[...] = (jnp.dot(h1, wskip_ref[...].T,
                              preferred_element_type=jnp.float32)
                      + bskip_ref[...][None])


def _k2(u0, u1, hlin, dinv, gcnb, g, b, wq, bq, wk, bk, wv, bv, wskip, bskip):
    sh = jax.ShapeDtypeStruct((_N, _C), jnp.float32)
    shb = jax.ShapeDtypeStruct((_N, _C), jnp.bfloat16)
    return pl.pallas_call(
        _k2_body,
        out_shape=(sh, shb, shb, sh),
    )(u0, u1, hlin, dinv, gcnb, g, b, wq, bq, wk, bk, wv, bv, wskip, bskip)


def _k3_body(ut0_ref, ut1_ref, ss0_ref, ss1_ref, hskip_ref,
             g_ref, b_ref, linw_ref, linb_ref, out_ref):
    ssum = ss0_ref[...] + ss1_ref[...]
    out_t = (ut0_ref[...] + ut1_ref[...]) / (ssum + 1e-16)[:, None]
    h2 = out_t + hskip_ref[...]
    h2 = _bn_rows(h2, g_ref[...], b_ref[...])
    out_ref[...] = (jnp.dot(h2, linw_ref[...].T,
                            preferred_element_type=jnp.float32)
                    + linb_ref[...][None])


def _k3(ut0, ut1, ss0, ss1, hskip, g, b, linw, linb):
    return pl.pallas_call(
        _k3_body,
        out_shape=jax.ShapeDtypeStruct((_N, _C), jnp.float32),
    )(ut0, ut1, ss0, ss1, hskip, g, b, linw, linb)


def kernel(x, edge_index, edge_weight, batch, nrows, ncols, conv2d_w, bn2d_g,
           bn2d_b, gcn_w, gcn_b, bn_g, bn_b, Wq, bq, Wk, bk, Wv, bv, We,
           Wskip, bskip, lin_w, lin_b):
    src = edge_index[0]
    dst = edge_index[1]

    npad_e = _EPAD - _E
    srcp = jnp.pad(src, (0, npad_e)).reshape(_ERW, _EW)
    dstp = jnp.pad(dst, (0, npad_e), constant_values=_N).reshape(_ERW, _EW)
    ewpad = jnp.pad(edge_weight, ((0, npad_e), (0, 0)))
    ew0p = ewpad[:, 0].reshape(_ERW, _EW)
    ew1p = ewpad[:, 1].reshape(_ERW, _EW)

    zerot = jnp.zeros((_NPAD, _C), jnp.float32)
    zerov = jnp.zeros((_NPAD,), jnp.float32)

    x2 = x + jnp.asarray(nrows * ncols - _N, jnp.float32)
    xpad = jnp.pad(x2.reshape(100, 100, _C), ((1, 1), (1, 1), (0, 0)))

    degb = _sca(dstp, ew1p, zerov)
    hlin, hlin2, dinv = _k1(xpad, conv2d_w, bn2d_g, bn2d_b, gcn_w,
                            degb[:_N], degb[_NPAD:_NPAD + _N])

    hlin2p = jnp.pad(hlin2, ((0, _NPAD - _N), (0, 0)))
    ub = _scb(srcp, dstp, ew1p, hlin2p, zerot)

    q, k, v, hskip = _k2(ub[0, :_N], ub[1, :_N], hlin, dinv, gcn_b,
                         bn_g, bn_b, Wq, bq, Wk, bk, Wv, bv, Wskip, bskip)

    qp = jnp.pad(q, ((0, _NPAD - _N), (0, 0)))
    kvp = lax.bitcast_convert_type(
        jnp.pad(jnp.concatenate([k, v], axis=1),
                ((0, _NPAD - _N), (0, 0))).reshape(_NPAD, _C, 2),
        jnp.int32)
    we0pk = lax.bitcast_convert_type(
        We[:, 0].astype(jnp.bfloat16).reshape(_C // 2, 2), jnp.int32)
    we1pk = lax.bitcast_convert_type(
        We[:, 1].astype(jnp.bfloat16).reshape(_C // 2, 2), jnp.int32)

    utb, ssb = _scc(srcp, dstp, ew0p, ew1p, qp, kvp, we0pk, we1pk,
                    zerot, zerov)

    return _k3(utb[0, :_N], utb[1, :_N], ssb[:_N], ssb[_NPAD:_NPAD + _N],
               hskip, bn_g, bn_b, lin_w, lin_b)


# speedup vs baseline: 3.5208x; 1.0240x over previous
import jax
import jax.numpy as jnp
from jax import lax
from jax.experimental import pallas as pl
from jax.experimental.pallas import tpu as pltpu
from jax.experimental.pallas import tpu_sc as plsc

_N = 10000
_C = 128
_RS = 0.08838834764831845

_NC, _NS, _L = 2, 16, 16
_NW = _NC * _NS
_NPAD = 10240
_E = 320000
_EW = 32
_ERW = 10240
_EPAD = _ERW * _EW
_ERT = _ERW // _NW
_NCH = 32
_NBLK = _ERT // _NCH
_ZS = _NPAD // _NS

_sc_mesh = plsc.VectorSubcoreMesh(core_axis_name="c", subcore_axis_name="s",
                                  num_cores=_NC, num_subcores=_NS)
_sc_params = pltpu.CompilerParams(needs_layout_passes=False)


def _elu(x):
    return jnp.where(x > 0, x, 0.1 * (jnp.exp(x) - 1.0))


def _bn_rows(h, g, b):
    mu = jnp.mean(h, axis=0, keepdims=True)
    var = jnp.mean((h - mu) ** 2, axis=0, keepdims=True)
    return (h - mu) / jnp.sqrt(var + 1e-5) * g[None] + b[None]


def _sca_body(dstp, ewp, zerov, out, idx_v, val_v, deg_sh):
    c = lax.axis_index("c")
    s = lax.axis_index("s")
    wid = s * _NC + c
    pltpu.sync_copy(zerov.at[pl.ds(s * _ZS, _ZS)],
                    deg_sh.at[pl.ds(s * _ZS, _ZS)])
    plsc.subcore_barrier()
    base = wid * _ERT

    def chunk(t, carry):
        pltpu.sync_copy(dstp.at[pl.ds(base + t * 32, 32)], idx_v)
        pltpu.sync_copy(ewp.at[pl.ds(base + t * 32, 32)], val_v)

        def row(j, cc):
            pltpu.sync_copy(val_v.at[j], deg_sh.at[idx_v.at[j]], add=True)
            return cc

        return lax.fori_loop(0, 32, row, carry)

    lax.fori_loop(0, _ERT // 32, chunk, 0)
    plsc.subcore_barrier()
    pltpu.sync_copy(deg_sh.at[pl.ds(s * _ZS, _ZS)],
                    out.at[pl.ds(c * _NPAD + s * _ZS, _ZS)])


def _sca(dstp, ewp, zerov):
    return pl.kernel(
        _sca_body,
        out_type=jax.ShapeDtypeStruct((_NC * _NPAD,), jnp.float32),
        mesh=_sc_mesh,
        scratch_types=[pltpu.VMEM((32, _EW), jnp.int32),
                       pltpu.VMEM((32, _EW), jnp.float32),
                       pltpu.VMEM_SHARED((_NPAD,), jnp.float32)],
        compiler_params=_sc_params,
    )(dstp, ewp, zerov)


def _scb_body(srcp, dstp, ewp, hlin2p, zerot, ub,
              src_b, dst_b, ew_b, rows0, rows1, vsb0, vsb1,
              gs0, gs1, ss0, ss1, acc_sh):
    c = lax.axis_index("c")
    s = lax.axis_index("s")
    wid = s * _NC + c
    rows = [rows0, rows1]
    vsb = [vsb0, vsb1]
    gsem = [gs0, gs1]
    ssem = [ss0, ss1]
    pltpu.sync_copy(zerot.at[pl.ds(s * _ZS, _ZS)],
                    acc_sh.at[pl.ds(s * _ZS, _ZS)])
    plsc.subcore_barrier()
    base = wid * _ERT
    iota = lax.iota(jnp.int32, 16)

    def block(lb, carry):
        r0 = base + lb * _NCH
        pltpu.sync_copy(srcp.at[pl.ds(r0, _NCH)], src_b)
        pltpu.sync_copy(dstp.at[pl.ds(r0, _NCH)], dst_b)
        pltpu.sync_copy(ewp.at[pl.ds(r0, _NCH)], ew_b)
        for b in range(2):
            pltpu.async_copy(hlin2p.at[src_b.at[b]], rows[b], gsem[b])

        def pair(cp, cc):
            for b in range(2):
                ch = cp * 2 + b
                pltpu.make_async_copy(hlin2p.at[src_b.at[ch]],
                                      rows[b], gsem[b]).wait()

                @pl.when(ch >= 2)
                def _():
                    pltpu.make_async_copy(
                        vsb[b], acc_sh.at[dst_b.at[ch]], ssem[b]).wait()

                ew = [ew_b[ch, pl.ds(g * 16, 16)] for g in range(2)]

                def feat(f, c2):
                    fv = jnp.full((16,), f, jnp.int32)
                    for g in range(2):
                        row16 = g * 16 + iota
                        hv = plsc.load_gather(rows[b], [row16, fv])
                        plsc.store_scatter(vsb[b], [row16, fv], hv * ew[g])
                    return c2

                lax.fori_loop(0, 128, feat, 0)
                pltpu.async_copy(vsb[b], acc_sh.at[dst_b.at[ch]],
                                 ssem[b], add=True)

                @pl.when(ch + 2 < _NCH)
                def _():
                    pltpu.async_copy(hlin2p.at[src_b.at[ch + 2]],
                                     rows[b], gsem[b])

            return cc

        lax.fori_loop(0, _NCH // 2, pair, 0)
        for b in range(2):
            pltpu.make_async_copy(vsb[b], acc_sh.at[dst_b.at[b]],
                                  ssem[b]).wait()
        return carry

    lax.fori_loop(0, _NBLK, block, 0)
    plsc.subcore_barrier()
    pltpu.sync_copy(acc_sh.at[pl.ds(s * _ZS, _ZS)],
                    ub.at[c, pl.ds(s * _ZS, _ZS)])


def _scb(srcp, dstp, ewp, hlin2p, zerot):
    return pl.kernel(
        _scb_body,
        out_type=jax.ShapeDtypeStruct((_NC, _NPAD, _C), jnp.float32),
        mesh=_sc_mesh,
        scratch_types=[pltpu.VMEM((_NCH, _EW), jnp.int32),
                       pltpu.VMEM((_NCH, _EW), jnp.int32),
                       pltpu.VMEM((_NCH, _EW), jnp.float32),
                       pltpu.VMEM((_EW, _C), jnp.float32),
                       pltpu.VMEM((_EW, _C), jnp.float32),
                       pltpu.VMEM((_EW, _C), jnp.float32),
                       pltpu.VMEM((_EW, _C), jnp.float32),
                       pltpu.SemaphoreType.DMA,
                       pltpu.SemaphoreType.DMA,
                       pltpu.SemaphoreType.DMA,
                       pltpu.SemaphoreType.DMA,
                       pltpu.VMEM_SHARED((_NPAD, _C), jnp.float32)],
        compiler_params=_sc_params,
    )(srcp, dstp, ewp, hlin2p, zerot)


def _scc_body(srcp, dstp, e0p, e1p, qp, kvp, we0in, we1in, zerot, zerov,
              utb, ssb,
              src_b, dst_b, e0_b, e1_b, qr0, qr1, kvr0, kvr1, vsb0, vsb1,
              ab0, ab1, we0_v, we1_v,
              gs0, gs1, ss0, ss1, acc_sh, ss_sh):
    c = lax.axis_index("c")
    s = lax.axis_index("s")
    wid = s * _NC + c
    qr = [qr0, qr1]
    kvr = [kvr0, kvr1]
    vsb = [vsb0, vsb1]
    ab = [ab0, ab1]
    gsem = [gs0, gs1]
    ssem = [ss0, ss1]
    pltpu.sync_copy(we0in, we0_v)
    pltpu.sync_copy(we1in, we1_v)
    pltpu.sync_copy(zerot.at[pl.ds(s * _ZS, _ZS)],
                    acc_sh.at[pl.ds(s * _ZS, _ZS)])
    pltpu.sync_copy(zerov.at[pl.ds(s * _ZS, _ZS)],
                    ss_sh.at[pl.ds(s * _ZS, _ZS)])
    plsc.subcore_barrier()
    base = wid * _ERT
    iota = lax.iota(jnp.int32, 16)
    z16 = jnp.zeros((16,), jnp.float32)

    def block(lb, carry):
        r0 = base + lb * _NCH
        pltpu.sync_copy(srcp.at[pl.ds(r0, _NCH)], src_b)
        pltpu.sync_copy(dstp.at[pl.ds(r0, _NCH)], dst_b)
        pltpu.sync_copy(e0p.at[pl.ds(r0, _NCH)], e0_b)
        pltpu.sync_copy(e1p.at[pl.ds(r0, _NCH)], e1_b)
        for b in range(2):
            pltpu.async_copy(qp.at[dst_b.at[b]], qr[b], gsem[b])
            pltpu.async_copy(kvp.at[src_b.at[b]], kvr[b], gsem[b])

        def pair(cp, cc):
            for b in range(2):
                ch = cp * 2 + b
                pltpu.make_async_copy(qp.at[dst_b.at[ch]],
                                      qr[b], gsem[b]).wait()
                pltpu.make_async_copy(kvp.at[src_b.at[ch]],
                                      kvr[b], gsem[b]).wait()
                ew0 = [e0_b[ch, pl.ds(g * 16, 16)] for g in range(2)]
                ew1 = [e1_b[ch, pl.ds(g * 16, 16)] for g in range(2)]

                def feat(f, al):
                    fv = jnp.full((16,), f, jnp.int32)
                    w0g = plsc.load_gather(we0_v, [fv])
                    w0l = plsc.bitcast(lax.shift_left(w0g, 16), jnp.float32)
                    w0h = plsc.bitcast(w0g & jnp.int32(-65536), jnp.float32)
                    w1g = plsc.load_gather(we1_v, [fv])
                    w1l = plsc.bitcast(lax.shift_left(w1g, 16), jnp.float32)
                    w1h = plsc.bitcast(w1g & jnp.int32(-65536), jnp.float32)
                    out = []
                    for g in range(2):
                        row16 = g * 16 + iota
                        ql = plsc.load_gather(qr[b], [row16, fv * 2])
                        qh = plsc.load_gather(qr[b], [row16, fv * 2 + 1])
                        kg = plsc.load_gather(kvr[b], [row16, fv])
                        kl = plsc.bitcast(lax.shift_left(kg, 16),
                                          jnp.float32)
                        kh = plsc.bitcast(kg & jnp.int32(-65536),
                                          jnp.float32)
                        out.append(al[g]
                                   + ql * (kl + w0l * ew0[g] + w1l * ew1[g])
                                   + qh * (kh + w0h * ew0[g] + w1h * ew1[g]))
                    return tuple(out)

                al = lax.fori_loop(0, 64, feat, (z16, z16))
                a16 = [jnp.exp(al[g] * _RS) for g in range(2)]

                @pl.when(ch >= 2)
                def _():
                    pltpu.make_async_copy(
                        vsb[b], acc_sh.at[dst_b.at[ch]], ssem[b]).wait()
                    pltpu.make_async_copy(
                        ab[b].at[0], ss_sh.at[dst_b.at[ch]], ssem[b]).wait()

                for g in range(2):
                    ab[b][0, pl.ds(g * 16, 16)] = a16[g]

                def feat2(f, c2):
                    fv = jnp.full((16,), f, jnp.int32)
                    w0g = plsc.load_gather(we0_v, [fv])
                    w0l = plsc.bitcast(lax.shift_left(w0g, 16), jnp.float32)
                    w0h = plsc.bitcast(w0g & jnp.int32(-65536), jnp.float32)
                    w1g = plsc.load_gather(we1_v, [fv])
                    w1l = plsc.bitcast(lax.shift_left(w1g, 16), jnp.float32)
                    w1h = plsc.bitcast(w1g & jnp.int32(-65536), jnp.float32)
                    for g in range(2):
                        row16 = g * 16 + iota
                        vg = plsc.load_gather(kvr[b], [row16, fv + 64])
                        vl = plsc.bitcast(lax.shift_left(vg, 16),
                                          jnp.float32)
                        vh = plsc.bitcast(vg & jnp.int32(-65536),
                                          jnp.float32)
                        plsc.store_scatter(
                            vsb[b], [row16, fv * 2],
                            (vl + w0l * ew0[g] + w1l * ew1[g]) * a16[g])
                        plsc.store_scatter(
                            vsb[b], [row16, fv * 2 + 1],
                            (vh + w0h * ew0[g] + w1h * ew1[g]) * a16[g])
                    return c2

                lax.fori_loop(0, 64, feat2, 0)
                pltpu.async_copy(vsb[b], acc_sh.at[dst_b.at[ch]],
                                 ssem[b], add=True)
                pltpu.async_copy(ab[b].at[0], ss_sh.at[dst_b.at[ch]],
                                 ssem[b], add=True)

                @pl.when(ch + 2 < _NCH)
                def _():
                    pltpu.async_copy(qp.at[dst_b.at[ch + 2]], qr[b], gsem[b])
                    pltpu.async_copy(kvp.at[src_b.at[ch + 2]], kvr[b],
                                     gsem[b])

            return cc

        lax.fori_loop(0, _NCH // 2, pair, 0)
        for b in range(2):
            pltpu.make_async_copy(vsb[b], acc_sh.at[dst_b.at[b]],
                                  ssem[b]).wait()
            pltpu.make_async_copy(ab[b].at[0], ss_sh.at[dst_b.at[b]],
                                  ssem[b]).wait()
        return carry

    lax.fori_loop(0, _NBLK, block, 0)
    plsc.subcore_barrier()
    pltpu.sync_copy(acc_sh.at[pl.ds(s * _ZS, _ZS)],
                    utb.at[c, pl.ds(s * _ZS, _ZS)])
    pltpu.sync_copy(ss_sh.at[pl.ds(s * _ZS, _ZS)],
                    ssb.at[pl.ds(c * _NPAD + s * _ZS, _ZS)])


def _scc(srcp, dstp, e0p, e1p, qp, kvp, we0in, we1in, zerot, zerov):
    return pl.kernel(
        _scc_body,
        out_type=(jax.ShapeDtypeStruct((_NC, _NPAD, _C), jnp.float32),
                  jax.ShapeDtypeStruct((_NC * _NPAD,), jnp.float32)),
        mesh=_sc_mesh,
        scratch_types=[pltpu.VMEM((_NCH, _EW), jnp.int32),
                       pltpu.VMEM((_NCH, _EW), jnp.int32),
                       pltpu.VMEM((_NCH, _EW), jnp.float32),
                       pltpu.VMEM((_NCH, _EW), jnp.float32),
                       pltpu.VMEM((_EW, _C), jnp.float32),
                       pltpu.VMEM((_EW, _C), jnp.float32),
                       pltpu.VMEM((_EW, _C), jnp.int32),
                       pltpu.VMEM((_EW, _C), jnp.int32),
                       pltpu.VMEM((_EW, _C), jnp.float32),
                       pltpu.VMEM((_EW, _C), jnp.float32),
                       pltpu.VMEM((1, _EW), jnp.float32),
                       pltpu.VMEM((1, _EW), jnp.float32),
                       pltpu.VMEM((_C // 2,), jnp.int32),
                       pltpu.VMEM((_C // 2,), jnp.int32),
                       pltpu.SemaphoreType.DMA,
                       pltpu.SemaphoreType.DMA,
                       pltpu.SemaphoreType.DMA,
                       pltpu.SemaphoreType.DMA,
                       pltpu.VMEM_SHARED((_NPAD, _C), jnp.float32),
                       pltpu.VMEM_SHARED((_NPAD,), jnp.float32)],
        compiler_params=_sc_params,
    )(srcp, dstp, e0p, e1p, qp, kvp, we0in, we1in, zerot, zerov)


def _k1_body(xpad_ref, w_ref, g_ref, b_ref, gcnw_ref, deg0_ref, deg1_ref,
             hlin_ref, hlin2_ref, dinv_ref):
    acc = jnp.zeros((_N, _C), jnp.float32)
    for di in range(3):
        for dj in range(3):
            xs = xpad_ref[di:di + 100, dj:dj + 100, :].reshape(_N, _C)
            acc = acc + jnp.dot(xs, w_ref[di, dj],
                                preferred_element_type=jnp.float32)
    h = _bn_rows(acc, g_ref[...], b_ref[...])
    h = _elu(h)
    hlin = jnp.dot(h, gcnw_ref[...].T, preferred_element_type=jnp.float32)
    dinv = lax.rsqrt(deg0_ref[...] + deg1_ref[...] + 2.0)
    hlin_ref[...] = hlin
    hlin2_ref[...] = dinv[:, None] * hlin
    dinv_ref[...] = dinv


def _k1(xpad, w, g, b, gcnw, deg0, deg1):
    sh = jax.ShapeDtypeStruct((_N, _C), jnp.float32)
    return pl.pallas_call(
        _k1_body,
        out_shape=(sh, sh, jax.ShapeDtypeStruct((_N,), jnp.float32)),
    )(xpad, w, g, b, gcnw, deg0, deg1)


def _k2_body(u0_ref, u1_ref, hlin_ref, dinv_ref, gcnb_ref, g_ref, b_ref,
             wq_ref, bq_ref, wk_ref, bk_ref, wv_ref, bv_ref,
             wskip_ref, bskip_ref,
             q_ref, k_ref, v_ref, hskip_ref):
    dinv = dinv_ref[...]
    h1 = (dinv[:, None] * (u0_ref[...] + u1_ref[...])
          + (2.0 * dinv * dinv)[:, None] * hlin_ref[...]
          + gcnb_ref[...][None])
    h1 = _elu(_bn_rows(h1, g_ref[...], b_ref[...]))
    q_ref[...] = jnp.dot(h1, wq_ref[...].T,
                         preferred_element_type=jnp.float32) + bq_ref[...][None]
    k_ref[...] = (jnp.dot(h1, wk_ref[...].T,
                          preferred_element_type=jnp.float32)
                  + bk_ref[...][None]).astype(jnp.bfloat16)
    v_ref[...] = (jnp.dot(h1, wv_ref[...].T,
                          preferred_element_type=jnp.float32)
                  + bv_ref[...][None]).astype(jnp.bfloat16)
    hskip_ref[...] = (jnp.dot(h1, wskip_ref[...].T,
                              preferred_element_type=jnp.float32)
                      + bskip_ref[...][None])


def _k2(u0, u1, hlin, dinv, gcnb, g, b, wq, bq, wk, bk, wv, bv, wskip, bskip):
    sh = jax.ShapeDtypeStruct((_N, _C), jnp.float32)
    shb = jax.ShapeDtypeStruct((_N, _C), jnp.bfloat16)
    return pl.pallas_call(
        _k2_body,
        out_shape=(sh, shb, shb, sh),
    )(u0, u1, hlin, dinv, gcnb, g, b, wq, bq, wk, bk, wv, bv, wskip, bskip)


def _k3_body(ut0_ref, ut1_ref, ss0_ref, ss1_ref, hskip_ref,
             g_ref, b_ref, linw_ref, linb_ref, out_ref):
    ssum = ss0_ref[...] + ss1_ref[...]
    out_t = (ut0_ref[...] + ut1_ref[...]) / (ssum + 1e-16)[:, None]
    h2 = out_t + hskip_ref[...]
    h2 = _bn_rows(h2, g_ref[...], b_ref[...])
    out_ref[...] = (jnp.dot(h2, linw_ref[...].T,
                            preferred_element_type=jnp.float32)
                    + linb_ref[...][None])


def _k3(ut0, ut1, ss0, ss1, hskip, g, b, linw, linb):
    return pl.pallas_call(
        _k3_body,
        out_shape=jax.ShapeDtypeStruct((_N, _C), jnp.float32),
    )(ut0, ut1, ss0, ss1, hskip, g, b, linw, linb)


def kernel(x, edge_index, edge_weight, batch, nrows, ncols, conv2d_w, bn2d_g,
           bn2d_b, gcn_w, gcn_b, bn_g, bn_b, Wq, bq, Wk, bk, Wv, bv, We,
           Wskip, bskip, lin_w, lin_b):
    src = edge_index[0]
    dst = edge_index[1]

    npad_e = _EPAD - _E
    srcp = jnp.pad(src, (0, npad_e)).reshape(_ERW, _EW)
    dstp = jnp.pad(dst, (0, npad_e), constant_values=_N).reshape(_ERW, _EW)
    ewpad = jnp.pad(edge_weight, ((0, npad_e), (0, 0)))
    ew0p = ewpad[:, 0].reshape(_ERW, _EW)
    ew1p = ewpad[:, 1].reshape(_ERW, _EW)

    zerot = jnp.zeros((_NPAD, _C), jnp.float32)
    zerov = jnp.zeros((_NPAD,), jnp.float32)

    x2 = x + jnp.asarray(nrows * ncols - _N, jnp.float32)
    xpad = jnp.pad(x2.reshape(100, 100, _C), ((1, 1), (1, 1), (0, 0)))

    degb = _sca(dstp, ew1p, zerov)
    hlin, hlin2, dinv = _k1(xpad, conv2d_w, bn2d_g, bn2d_b, gcn_w,
                            degb[:_N], degb[_NPAD:_NPAD + _N])

    hlin2p = jnp.pad(hlin2, ((0, _NPAD - _N), (0, 0)))
    ub = _scb(srcp, dstp, ew1p, hlin2p, zerot)

    q, k, v, hskip = _k2(ub[0, :_N], ub[1, :_N], hlin, dinv, gcn_b,
                         bn_g, bn_b, Wq, bq, Wk, bk, Wv, bv, Wskip, bskip)

    qp = jnp.pad(q, ((0, _NPAD - _N), (0, 0)))
    kvp = lax.bitcast_convert_type(
        jnp.pad(jnp.concatenate([k, v], axis=1),
                ((0, _NPAD - _N), (0, 0))).reshape(_NPAD, _C, 2),
        jnp.int32)
    we0pk = lax.bitcast_convert_type(
        We[:, 0].astype(jnp.bfloat16).reshape(_C // 2, 2), jnp.int32)
    we1pk = lax.bitcast_convert_type(
        We[:, 1].astype(jnp.bfloat16).reshape(_C // 2, 2), jnp.int32)

    utb, ssb = _scc(srcp, dstp, ew0p, ew1p, qp, kvp, we0pk, we1pk,
                    zerot, zerov)

    return _k3(utb[0, :_N], utb[1, :_N], ssb[:_N], ssb[_NPAD:_NPAD + _N],
               hskip, bn_g, bn_b, lin_w, lin_b)
